# baseline (speedup 1.0000x reference)
import functools

import jax
import jax.numpy as jnp
from jax import lax
from jax.experimental import pallas as pl
from jax.experimental.pallas import tpu as pltpu
from jax.experimental.pallas import tpu_sc as plsc

_NUM = 1000000
_DIM = 32
_BATCH = 16384
_HIST = 50
_B = _BATCH * _HIST

_NC = 2
_NS = 16
_NW = _NC * _NS

_PER_W = _B // _NW
_GROUP = 1280
_N_GROUPS = _PER_W // _GROUP
_NPAIRS = _N_GROUPS // 2


@functools.partial(
    pl.kernel,
    mesh=plsc.VectorSubcoreMesh(core_axis_name="c", subcore_axis_name="s"),
    out_type=jax.ShapeDtypeStruct((_B, _DIM), jnp.float32),
    scratch_types=[
        pltpu.VMEM((_GROUP,), jnp.int32),
        pltpu.VMEM((_GROUP,), jnp.int32),
        pltpu.VMEM((_GROUP, _DIM), jnp.float32),
        pltpu.VMEM((_GROUP, _DIM), jnp.float32),
        pltpu.SemaphoreType.DMA,
        pltpu.SemaphoreType.DMA,
        pltpu.SemaphoreType.DMA,
        pltpu.SemaphoreType.DMA,
    ],
    compiler_params=pltpu.CompilerParams(use_tc_tiling_on_sc=False),
)
def _embed_gather(idx_hbm, table_hbm, out_hbm,
                  idx_v0, idx_v1, rows_v0, rows_v1,
                  gsem0, gsem1, wsem0, wsem1):
    wid = lax.axis_index("s") * _NC + lax.axis_index("c")
    w_base = wid * _PER_W
    idx_v = (idx_v0, idx_v1)
    rows_v = (rows_v0, rows_v1)
    gsem = (gsem0, gsem1)
    wsem = (wsem0, wsem1)

    def fire(b, base):
        pltpu.sync_copy(idx_hbm.at[pl.ds(base, _GROUP)], idx_v[b])
        pltpu.async_copy(table_hbm.at[idx_v[b]], rows_v[b], gsem[b])

    def drain(b):
        pltpu.make_async_copy(
            out_hbm.at[pl.ds(0, _GROUP)], rows_v[b], gsem[b]
        ).wait()

    for b in range(2):
        fire(b, w_base + b * _GROUP)

    def pair_body(i, carry):
        for b in range(2):
            base = w_base + (2 * i + b) * _GROUP
            drain(b)
            pltpu.async_copy(
                rows_v[b], out_hbm.at[pl.ds(base, _GROUP)], wsem[b]
            ).wait()
            fire(b, base + 2 * _GROUP)
        return carry

    lax.fori_loop(0, _NPAIRS - 1, pair_body, 0)

    for b in range(2):
        base = w_base + (_N_GROUPS - 2 + b) * _GROUP
        drain(b)
        pltpu.async_copy(
            rows_v[b], out_hbm.at[pl.ds(base, _GROUP)], wsem[b]
        ).wait()


def kernel(idx_list, table):
    idx_flat = idx_list.reshape(_B)
    out = _embed_gather(idx_flat, table)
    return out.reshape(1, _BATCH, _HIST, _DIM)


# speedup vs baseline: 1.6251x; 1.6251x over previous
import functools

import jax
import jax.numpy as jnp
from jax import lax
from jax.experimental import pallas as pl
from jax.experimental.pallas import tpu as pltpu
from jax.experimental.pallas import tpu_sc as plsc

_NUM = 1000000
_DIM = 32
_BATCH = 16384
_HIST = 50

_NC = 2
_NS = 16
_NW = _NC * _NS

_ROWS_PER_W = _BATCH // _NW
_GROUP_ROWS = 32
_N_GROUPS = _ROWS_PER_W // _GROUP_ROWS
_NPAIRS = _N_GROUPS // 2


@functools.partial(
    pl.kernel,
    mesh=plsc.VectorSubcoreMesh(core_axis_name="c", subcore_axis_name="s"),
    out_type=jax.ShapeDtypeStruct((1, _BATCH, _HIST, _DIM), jnp.float32),
    scratch_types=[
        pltpu.VMEM((_GROUP_ROWS, _HIST), jnp.int32),
        pltpu.VMEM((_GROUP_ROWS, _HIST), jnp.int32),
        pltpu.VMEM((_GROUP_ROWS, _HIST, _DIM), jnp.float32),
        pltpu.VMEM((_GROUP_ROWS, _HIST, _DIM), jnp.float32),
        pltpu.SemaphoreType.DMA,
        pltpu.SemaphoreType.DMA,
        pltpu.SemaphoreType.DMA,
        pltpu.SemaphoreType.DMA,
    ],
    compiler_params=pltpu.CompilerParams(use_tc_tiling_on_sc=False),
)
def _embed_gather(idx_hbm, table_hbm, out_hbm,
                  idx_v0, idx_v1, rows_v0, rows_v1,
                  gsem0, gsem1, wsem0, wsem1):
    wid = lax.axis_index("s") * _NC + lax.axis_index("c")
    w_row0 = wid * _ROWS_PER_W
    idx_v = (idx_v0, idx_v1)
    rows_v = (rows_v0, rows_v1)
    gsem = (gsem0, gsem1)
    wsem = (wsem0, wsem1)

    def fire(b, row0):
        pltpu.sync_copy(idx_hbm.at[pl.ds(row0, _GROUP_ROWS)], idx_v[b])
        for j in range(_GROUP_ROWS):
            pltpu.async_copy(
                table_hbm.at[idx_v[b].at[j]], rows_v[b].at[j], gsem[b]
            )

    def drain(b):
        pltpu.make_async_copy(
            out_hbm.at[0, pl.ds(0, _GROUP_ROWS)], rows_v[b], gsem[b]
        ).wait()

    for b in range(2):
        fire(b, w_row0 + b * _GROUP_ROWS)

    def pair_body(i, carry):
        for b in range(2):
            row0 = w_row0 + (2 * i + b) * _GROUP_ROWS
            drain(b)
            pltpu.async_copy(
                rows_v[b], out_hbm.at[0, pl.ds(row0, _GROUP_ROWS)], wsem[b]
            ).wait()
            fire(b, row0 + 2 * _GROUP_ROWS)
        return carry

    lax.fori_loop(0, _NPAIRS - 1, pair_body, 0)

    for b in range(2):
        row0 = w_row0 + (_N_GROUPS - 2 + b) * _GROUP_ROWS
        drain(b)
        pltpu.async_copy(
            rows_v[b], out_hbm.at[0, pl.ds(row0, _GROUP_ROWS)], wsem[b]
        ).wait()


def kernel(idx_list, table):
    return _embed_gather(idx_list, table)
